# baseline (speedup 1.0000x reference)
import functools

import jax
import jax.numpy as jnp
from jax import lax
from jax.experimental import pallas as pl
from jax.experimental.pallas import tpu as pltpu
from jax.experimental.pallas import tpu_sc as plsc

_N = 10000
_FDIM = 256
_E = 160000

_BN = 1024
_NPAD = 10240
_NC, _NS = 2, 16
_NW = _NC * _NS
_SEG = _NPAD // _NS
_C1 = 5008
_C1L = _E - 31 * _C1
_W1 = 5120
_EPT2 = _E // _NS
_W2 = 10112


def _b_body(x_ref, w_ref, bias_ref, out_ref):
    h = lax.dot_general(w_ref[...], x_ref[...], (((1,), (1,)), ((), ())),
                        preferred_element_type=jnp.float32)
    h = jnp.maximum(h + bias_ref[...].T, 0.0)

    ones = jnp.ones((1, _FDIM), jnp.float32)
    rs = lax.dot_general(ones, h, (((1,), (0,)), ((), ())),
                         preferred_element_type=jnp.float32)
    col = pl.program_id(0) * _BN + lax.broadcasted_iota(jnp.int32, (1, _BN), 1)
    out_ref[...] = jnp.where(col < _N, rs, 0.0).reshape(_BN)


def _node_sums(x, w, bias2):
    return pl.pallas_call(
        _b_body,
        grid=(_NPAD // _BN,),
        in_specs=[
            pl.BlockSpec((_BN, _FDIM), lambda i: (i, 0)),
            pl.BlockSpec((_FDIM, _FDIM), lambda i: (0, 0)),
            pl.BlockSpec((1, _FDIM), lambda i: (0, 0)),
        ],
        out_specs=pl.BlockSpec((_BN,), lambda i: (i,)),
        out_shape=jax.ShapeDtypeStruct((_NPAD,), jnp.float32),
    )(x, w, bias2)


def _sc_body(b_hbm, ei_hbm, out_hbm,
             b_v, aux_v, e1_v, e2_v, acc_v,
             gbuf_v, seg_v, all_sh, merged_sh, sem1, sem2, semm):
    cid = lax.axis_index("c")
    sid = lax.axis_index("s")
    wid = cid * _NS + sid

    start1 = wid * _C1
    a1 = pl.multiple_of(
        jnp.where(wid == _NW - 1, _E - _W1, start1 - start1 % 128), 128)
    o1 = start1 - a1
    start2 = sid * _EPT2
    a2 = pl.multiple_of(start2 - start2 % 128, 128)
    o2 = start2 - a2
    cps = [
        pltpu.async_copy(b_hbm, b_v, sem1),
        pltpu.async_copy(ei_hbm.at[:, pl.ds(a1, _W1)], e1_v, sem1),
    ]
    cp2 = [
        pltpu.async_copy(ei_hbm.at[:, pl.ds(a2, _W2)], e2_v, sem2),
    ]

    @plsc.parallel_loop(0, _NPAD // 16, unroll=8)
    def zero_step(i):
        aux_v[pl.ds(i * 16, 16)] = jnp.zeros((16,), jnp.float32)

    for cp in cps:
        cp.wait()

    @plsc.parallel_loop(0, _C1L // 16, unroll=8)
    def r1_step(i):
        s = e1_v[0, pl.ds(o1 + i * 16, 16)]
        d = e1_v[1, pl.ds(o1 + i * 16, 16)]
        vals = plsc.load_gather(b_v, [d])
        plsc.addupdate_scatter(aux_v, [s], vals)

    @pl.when(wid < _NW - 1)
    def _r1_rest():
        @plsc.parallel_loop(_C1L // 16, _C1 // 16, unroll=8)
        def r1_step2(i):
            s = e1_v[0, pl.ds(o1 + i * 16, 16)]
            d = e1_v[1, pl.ds(o1 + i * 16, 16)]
            vals = plsc.load_gather(b_v, [d])
            plsc.addupdate_scatter(aux_v, [s], vals)

    pltpu.sync_copy(aux_v, all_sh.at[sid])
    plsc.subcore_barrier()

    base = sid * _SEG
    cpm = [pltpu.async_copy(all_sh.at[t, pl.ds(base, _SEG)],
                            gbuf_v.at[t], semm)
           for t in range(_NS)]
    for cp in cpm:
        cp.wait()

    @plsc.parallel_loop(0, _SEG // 16, unroll=4)
    def add_step(j):
        acc = gbuf_v[0, pl.ds(j * 16, 16)]
        for t in range(1, _NS):
            acc = acc + gbuf_v[t, pl.ds(j * 16, 16)]
        seg_v[pl.ds(j * 16, 16)] = acc

    pltpu.sync_copy(seg_v, merged_sh.at[pl.ds(base, _SEG)])
    plsc.subcore_barrier()
    pltpu.sync_copy(merged_sh, aux_v)

    @plsc.parallel_loop(0, _NPAD // 16, unroll=8)
    def s1_step(i):
        aux_v[pl.ds(i * 16, 16)] = (aux_v[pl.ds(i * 16, 16)]
                                    * b_v[pl.ds(i * 16, 16)])

    for cp in cp2:
        cp.wait()

    @plsc.parallel_loop(0, _EPT2 // 16, unroll=16,
                        carry=jnp.zeros((16,), jnp.float32))
    def r2_step(i, acc):
        s = e2_v[0, pl.ds(o2 + i * 16, 16)]
        d = e2_v[1, pl.ds(o2 + i * 16, 16)]
        bs = plsc.load_gather(b_v, [s])
        sd = plsc.load_gather(aux_v, [d])
        return acc + bs * sd

    acc = r2_step
    acc_v[...] = acc
    pltpu.sync_copy(acc_v, out_hbm.at[wid])


@functools.cache
def _sc_call():
    return pl.kernel(
        _sc_body,
        out_type=jax.ShapeDtypeStruct((_NW, 16), jnp.float32),
        mesh=plsc.VectorSubcoreMesh(core_axis_name="c", subcore_axis_name="s",
                                    num_cores=_NC, num_subcores=_NS),
        compiler_params=pltpu.CompilerParams(needs_layout_passes=False),
        scratch_types=[
            pltpu.VMEM((_NPAD,), jnp.float32),
            pltpu.VMEM((_NPAD,), jnp.float32),
            pltpu.VMEM((2, _W1), jnp.int32),
            pltpu.VMEM((2, _W2), jnp.int32),
            pltpu.VMEM((16,), jnp.float32),
            pltpu.VMEM((_NS, _SEG), jnp.float32),
            pltpu.VMEM((_SEG,), jnp.float32),
            pltpu.VMEM_SHARED((_NS, _NPAD), jnp.float32),
            pltpu.VMEM_SHARED((_NPAD,), jnp.float32),
            pltpu.SemaphoreType.DMA,
            pltpu.SemaphoreType.DMA,
            pltpu.SemaphoreType.DMA,
        ],
    )


def kernel(X, edge_index, W, bias):
    b = _node_sums(X, W, bias.reshape(1, _FDIM))
    parts = _sc_call()(b, edge_index)
    return jnp.sum(parts)


# speedup vs baseline: 1.0235x; 1.0235x over previous
import functools

import jax
import jax.numpy as jnp
from jax import lax
from jax.experimental import pallas as pl
from jax.experimental.pallas import tpu as pltpu
from jax.experimental.pallas import tpu_sc as plsc

_N = 10000
_FDIM = 256
_E = 160000

_BN = 1024
_NPAD = 10240
_NC, _NS = 2, 16
_NW = _NC * _NS
_SEG = _NPAD // _NS
_C1 = 5008
_C1L = _E - 31 * _C1
_W1 = 5120
_EPT2 = _E // _NS
_W2 = 10112


def _b_body(x_ref, w_ref, bias_ref, out_ref):
    h = lax.dot_general(w_ref[...], x_ref[...], (((1,), (1,)), ((), ())),
                        preferred_element_type=jnp.float32)
    h = jnp.maximum(h + bias_ref[...].T, 0.0)

    ones = jnp.ones((1, _FDIM), jnp.float32)
    rs = lax.dot_general(ones, h, (((1,), (0,)), ((), ())),
                         preferred_element_type=jnp.float32)
    col = pl.program_id(0) * _BN + lax.broadcasted_iota(jnp.int32, (1, _BN), 1)
    out_ref[...] = jnp.where(col < _N, rs, 0.0).reshape(_BN)


def _node_sums(x, w, bias2):
    return pl.pallas_call(
        _b_body,
        grid=(_NPAD // _BN,),
        in_specs=[
            pl.BlockSpec((_BN, _FDIM), lambda i: (i, 0)),
            pl.BlockSpec((_FDIM, _FDIM), lambda i: (0, 0)),
            pl.BlockSpec((1, _FDIM), lambda i: (0, 0)),
        ],
        out_specs=pl.BlockSpec((_BN,), lambda i: (i,)),
        out_shape=jax.ShapeDtypeStruct((_NPAD,), jnp.float32),
    )(x, w, bias2)


def _sc_body(b_hbm, ei_hbm, out_hbm,
             b_v, aux_v, e1_v, e2_v, acc_v,
             gbuf_v, seg_v, all_sh, merged_sh, sem1, sem2, semm):
    cid = lax.axis_index("c")
    sid = lax.axis_index("s")
    wid = cid * _NS + sid

    start1 = wid * _C1
    a1 = pl.multiple_of(
        jnp.where(wid == _NW - 1, _E - _W1, start1 - start1 % 128), 128)
    o1 = start1 - a1
    start2 = sid * _EPT2
    a2 = pl.multiple_of(start2 - start2 % 128, 128)
    o2 = start2 - a2
    cps = [
        pltpu.async_copy(b_hbm, b_v, sem1),
        pltpu.async_copy(ei_hbm.at[:, pl.ds(a1, _W1)], e1_v, sem1),
    ]
    cp2 = [
        pltpu.async_copy(ei_hbm.at[:, pl.ds(a2, _W2)], e2_v, sem2),
    ]

    @plsc.parallel_loop(0, _NPAD // 16, unroll=8)
    def zero_step(i):
        aux_v[pl.ds(i * 16, 16)] = jnp.zeros((16,), jnp.float32)

    for cp in cps:
        cp.wait()

    @plsc.parallel_loop(0, _C1L // 16, unroll=8)
    def r1_step(i):
        s = e1_v[0, pl.ds(o1 + i * 16, 16)]
        d = e1_v[1, pl.ds(o1 + i * 16, 16)]
        vals = plsc.load_gather(b_v, [d])
        plsc.addupdate_scatter(aux_v, [s], vals)

    @pl.when(wid < _NW - 1)
    def _r1_rest():
        @plsc.parallel_loop(_C1L // 16, _C1 // 16, unroll=8)
        def r1_step2(i):
            s = e1_v[0, pl.ds(o1 + i * 16, 16)]
            d = e1_v[1, pl.ds(o1 + i * 16, 16)]
            vals = plsc.load_gather(b_v, [d])
            plsc.addupdate_scatter(aux_v, [s], vals)

    pltpu.sync_copy(aux_v, all_sh.at[sid])
    plsc.subcore_barrier()

    base = sid * _SEG
    cpm = [pltpu.async_copy(all_sh.at[t, pl.ds(base, _SEG)],
                            gbuf_v.at[t], semm)
           for t in range(_NS)]
    for cp in cpm:
        cp.wait()

    @plsc.parallel_loop(0, _SEG // 16, unroll=4)
    def add_step(j):
        acc = gbuf_v[0, pl.ds(j * 16, 16)]
        for t in range(1, _NS):
            acc = acc + gbuf_v[t, pl.ds(j * 16, 16)]
        seg_v[pl.ds(j * 16, 16)] = acc * b_v[pl.ds(base + j * 16, 16)]

    pltpu.sync_copy(seg_v, merged_sh.at[pl.ds(base, _SEG)])
    plsc.subcore_barrier()
    pltpu.sync_copy(merged_sh, aux_v)

    for cp in cp2:
        cp.wait()

    @plsc.parallel_loop(0, _EPT2 // 16, unroll=8,
                        carry=jnp.zeros((16,), jnp.float32))
    def r2_step(i, acc):
        s = e2_v[0, pl.ds(o2 + i * 16, 16)]
        d = e2_v[1, pl.ds(o2 + i * 16, 16)]
        bs = plsc.load_gather(b_v, [s])
        sd = plsc.load_gather(aux_v, [d])
        return acc + bs * sd

    acc = r2_step
    acc_v[...] = acc
    pltpu.sync_copy(acc_v, out_hbm.at[wid])


@functools.cache
def _sc_call():
    return pl.kernel(
        _sc_body,
        out_type=jax.ShapeDtypeStruct((_NW, 16), jnp.float32),
        mesh=plsc.VectorSubcoreMesh(core_axis_name="c", subcore_axis_name="s",
                                    num_cores=_NC, num_subcores=_NS),
        compiler_params=pltpu.CompilerParams(needs_layout_passes=False),
        scratch_types=[
            pltpu.VMEM((_NPAD,), jnp.float32),
            pltpu.VMEM((_NPAD,), jnp.float32),
            pltpu.VMEM((2, _W1), jnp.int32),
            pltpu.VMEM((2, _W2), jnp.int32),
            pltpu.VMEM((16,), jnp.float32),
            pltpu.VMEM((_NS, _SEG), jnp.float32),
            pltpu.VMEM((_SEG,), jnp.float32),
            pltpu.VMEM_SHARED((_NS, _NPAD), jnp.float32),
            pltpu.VMEM_SHARED((_NPAD,), jnp.float32),
            pltpu.SemaphoreType.DMA,
            pltpu.SemaphoreType.DMA,
            pltpu.SemaphoreType.DMA,
        ],
    )


def kernel(X, edge_index, W, bias):
    b = _node_sums(X, W, bias.reshape(1, _FDIM))
    parts = _sc_call()(b, edge_index)
    return jnp.sum(parts)


# speedup vs baseline: 1.1034x; 1.0781x over previous
import functools

import jax
import jax.numpy as jnp
from jax import lax
from jax.experimental import pallas as pl
from jax.experimental.pallas import tpu as pltpu
from jax.experimental.pallas import tpu_sc as plsc

_N = 10000
_FDIM = 256
_E = 160000

_BN = 2048
_NPAD = 10240
_NC, _NS = 2, 16
_NW = _NC * _NS
_SEG = _NPAD // _NS
_C1 = 5008
_C1L = _E - 31 * _C1
_W1 = 5120
_EPT2 = _E // _NS
_W2 = 10112


def _b_body(x_ref, w_ref, bias_ref, out_ref):
    h = lax.dot_general(w_ref[...], x_ref[...], (((1,), (1,)), ((), ())),
                        preferred_element_type=jnp.float32)
    h = jnp.maximum(h + bias_ref[...].T, 0.0)

    ones = jnp.ones((1, _FDIM), jnp.float32)
    rs = lax.dot_general(ones, h, (((1,), (0,)), ((), ())),
                         preferred_element_type=jnp.float32)
    col = pl.program_id(0) * _BN + lax.broadcasted_iota(jnp.int32, (1, _BN), 1)
    out_ref[...] = jnp.where(col < _N, rs, 0.0).reshape(_BN)


def _node_sums(x, w, bias2):
    return pl.pallas_call(
        _b_body,
        grid=(_NPAD // _BN,),
        in_specs=[
            pl.BlockSpec((_BN, _FDIM), lambda i: (i, 0)),
            pl.BlockSpec((_FDIM, _FDIM), lambda i: (0, 0)),
            pl.BlockSpec((1, _FDIM), lambda i: (0, 0)),
        ],
        out_specs=pl.BlockSpec((_BN,), lambda i: (i,)),
        out_shape=jax.ShapeDtypeStruct((_NPAD,), jnp.float32),
    )(x, w, bias2)


def _sc_body(b_hbm, ei_hbm, out_hbm,
             b_v, aux_v, e1_v, e2_v, acc_v,
             gbuf_v, seg_v, all_sh, merged_sh, sem1, sem2, semm):
    cid = lax.axis_index("c")
    sid = lax.axis_index("s")
    wid = cid * _NS + sid

    start1 = wid * _C1
    a1 = pl.multiple_of(
        jnp.where(wid == _NW - 1, _E - _W1, start1 - start1 % 128), 128)
    o1 = start1 - a1
    start2 = sid * _EPT2
    a2 = pl.multiple_of(start2 - start2 % 128, 128)
    o2 = start2 - a2
    cps = [
        pltpu.async_copy(b_hbm, b_v, sem1),
        pltpu.async_copy(ei_hbm.at[:, pl.ds(a1, _W1)], e1_v, sem1),
    ]
    cp2 = [
        pltpu.async_copy(ei_hbm.at[:, pl.ds(a2, _W2)], e2_v, sem2),
    ]

    @plsc.parallel_loop(0, _NPAD // 16, unroll=8)
    def zero_step(i):
        aux_v[pl.ds(i * 16, 16)] = jnp.zeros((16,), jnp.float32)

    for cp in cps:
        cp.wait()

    @plsc.parallel_loop(0, _C1L // 16, unroll=8)
    def r1_step(i):
        s = e1_v[0, pl.ds(o1 + i * 16, 16)]
        d = e1_v[1, pl.ds(o1 + i * 16, 16)]
        vals = plsc.load_gather(b_v, [d])
        plsc.addupdate_scatter(aux_v, [s], vals)

    @pl.when(wid < _NW - 1)
    def _r1_rest():
        @plsc.parallel_loop(_C1L // 16, _C1 // 16, unroll=8)
        def r1_step2(i):
            s = e1_v[0, pl.ds(o1 + i * 16, 16)]
            d = e1_v[1, pl.ds(o1 + i * 16, 16)]
            vals = plsc.load_gather(b_v, [d])
            plsc.addupdate_scatter(aux_v, [s], vals)

    pltpu.sync_copy(aux_v, all_sh.at[sid])
    plsc.subcore_barrier()

    base = sid * _SEG
    cpm = [pltpu.async_copy(all_sh.at[t, pl.ds(base, _SEG)],
                            gbuf_v.at[t], semm)
           for t in range(_NS)]
    for cp in cpm:
        cp.wait()

    @plsc.parallel_loop(0, _SEG // 16, unroll=4)
    def add_step(j):
        acc = gbuf_v[0, pl.ds(j * 16, 16)]
        for t in range(1, _NS):
            acc = acc + gbuf_v[t, pl.ds(j * 16, 16)]
        seg_v[pl.ds(j * 16, 16)] = acc * b_v[pl.ds(base + j * 16, 16)]

    pltpu.sync_copy(seg_v, merged_sh.at[pl.ds(base, _SEG)])
    plsc.subcore_barrier()
    pltpu.sync_copy(merged_sh, aux_v)

    for cp in cp2:
        cp.wait()

    @plsc.parallel_loop(0, _EPT2 // 16, unroll=8,
                        carry=jnp.zeros((16,), jnp.float32))
    def r2_step(i, acc):
        s = e2_v[0, pl.ds(o2 + i * 16, 16)]
        d = e2_v[1, pl.ds(o2 + i * 16, 16)]
        bs = plsc.load_gather(b_v, [s])
        sd = plsc.load_gather(aux_v, [d])
        return acc + bs * sd

    acc = r2_step
    acc_v[...] = acc
    pltpu.sync_copy(acc_v, out_hbm.at[wid])


@functools.cache
def _sc_call():
    return pl.kernel(
        _sc_body,
        out_type=jax.ShapeDtypeStruct((_NW, 16), jnp.float32),
        mesh=plsc.VectorSubcoreMesh(core_axis_name="c", subcore_axis_name="s",
                                    num_cores=_NC, num_subcores=_NS),
        compiler_params=pltpu.CompilerParams(needs_layout_passes=False),
        scratch_types=[
            pltpu.VMEM((_NPAD,), jnp.float32),
            pltpu.VMEM((_NPAD,), jnp.float32),
            pltpu.VMEM((2, _W1), jnp.int32),
            pltpu.VMEM((2, _W2), jnp.int32),
            pltpu.VMEM((16,), jnp.float32),
            pltpu.VMEM((_NS, _SEG), jnp.float32),
            pltpu.VMEM((_SEG,), jnp.float32),
            pltpu.VMEM_SHARED((_NS, _NPAD), jnp.float32),
            pltpu.VMEM_SHARED((_NPAD,), jnp.float32),
            pltpu.SemaphoreType.DMA,
            pltpu.SemaphoreType.DMA,
            pltpu.SemaphoreType.DMA,
        ],
    )


def kernel(X, edge_index, W, bias):
    b = _node_sums(X, W, bias.reshape(1, _FDIM))
    parts = _sc_call()(b, edge_index)
    return jnp.sum(parts)
